# baseline (speedup 1.0000x reference)
import functools

import jax
import jax.numpy as jnp
from jax import lax
from jax.experimental import pallas as pl
from jax.experimental.pallas import tpu as pltpu
from jax.experimental.pallas import tpu_sc as plsc

NC = 2
NS = 16
NW = NC * NS

_EC = 40
_K = 5
_PK = _K * _EC
_CAP = 10208


def _pad_rows(n):
  per = -(-n // NS)
  per = -(-per // 16) * 16
  return per * NS


@functools.lru_cache(maxsize=None)
def _make_sc_part(E, N):
  NP = _pad_rows(N)
  EW = E // NW
  CH = 2000
  NST = EW // CH
  mesh = plsc.VectorSubcoreMesh(core_axis_name="c", subcore_axis_name="s")

  @functools.partial(
      pl.kernel,
      mesh=mesh,
      compiler_params=pltpu.CompilerParams(needs_layout_passes=False),
      out_type=[
          jax.ShapeDtypeStruct((2 * NW * _CAP,), jnp.int32),
          jax.ShapeDtypeStruct((2 * NW * _CAP,), jnp.int32),
          jax.ShapeDtypeStruct((2 * NW * 16,), jnp.int32),
      ],
      scratch_types=[
          pltpu.VMEM((CH,), jnp.int32),
          pltpu.VMEM((CH,), jnp.int32),
          pltpu.VMEM((_CAP,), jnp.int32),
          pltpu.VMEM((_CAP,), jnp.int32),
          pltpu.VMEM((_CAP,), jnp.int32),
          pltpu.VMEM((_CAP,), jnp.int32),
          pltpu.VMEM((16,), jnp.int32),
      ],
  )
  def part_kernel(src_hbm, dst_hbm, gidx_hbm, gdst_hbm, cnt_hbm,
                  sbuf, dbuf, evs, evd, ods, odd, cbuf):
    cid = lax.axis_index("c")
    sid = lax.axis_index("s")
    wid = sid * NC + cid
    iota = lax.iota(jnp.int32, 16)
    zero = jnp.zeros((16,), jnp.int32)

    def stage(s, carry):
      ecnt, ocnt = carry
      base = wid * EW + s * CH
      pltpu.sync_copy(src_hbm.at[pl.ds(base, CH)], sbuf)
      pltpu.sync_copy(dst_hbm.at[pl.ds(base, CH)], dbuf)

      def vreg(i, c2):
        e2, o2 = c2
        sv = sbuf[pl.ds(i * 16, 16)]
        dv = dbuf[pl.ds(i * 16, 16)]
        r = lax.shift_right_logical(sv, 1)
        mi = 1 - (sv & 1)
        m = mi == 1
        moi = sv & 1
        mo = moi == 1
        pe = plsc.cumsum(mi) - mi + e2
        plsc.store_scatter(evs, [pe], r, mask=m)
        plsc.store_scatter(evd, [pe], dv, mask=m)
        po = plsc.cumsum(moi) - moi + o2
        plsc.store_scatter(ods, [po], r, mask=mo)
        plsc.store_scatter(odd, [po], dv, mask=mo)
        return (e2 + plsc.all_reduce_population_count(m),
                o2 + plsc.all_reduce_population_count(mo))

      return lax.fori_loop(0, CH // 16, vreg, (ecnt, ocnt))

    ecnt, ocnt = lax.fori_loop(0, NST, stage, (zero, zero))
    dr = (N // 2) + (iota & 7)
    dd = (NP - 8) + (iota & 7)
    for j in range(_PK // 16 + 1):
      plsc.store_scatter(evs, [iota + j * 16 + ecnt], dr)
      plsc.store_scatter(evd, [iota + j * 16 + ecnt], dd)
      plsc.store_scatter(ods, [iota + j * 16 + ocnt], dr)
      plsc.store_scatter(odd, [iota + j * 16 + ocnt], dd)
    ecp = (ecnt + _PK - 1) // _PK * _PK
    ocp = (ocnt + _PK - 1) // _PK * _PK
    cbuf[pl.ds(0, 16)] = ecp
    pltpu.sync_copy(cbuf, cnt_hbm.at[pl.ds(wid * 16, 16)])
    cbuf[pl.ds(0, 16)] = ocp
    pltpu.sync_copy(cbuf, cnt_hbm.at[pl.ds((NW + wid) * 16, 16)])
    pltpu.sync_copy(evs, gidx_hbm.at[pl.ds(wid * _CAP, _CAP)])
    pltpu.sync_copy(evd, gdst_hbm.at[pl.ds(wid * _CAP, _CAP)])
    pltpu.sync_copy(ods, gidx_hbm.at[pl.ds((NW + wid) * _CAP, _CAP)])
    pltpu.sync_copy(odd, gdst_hbm.at[pl.ds((NW + wid) * _CAP, _CAP)])

  return part_kernel


@functools.lru_cache(maxsize=None)
def _make_sc_agg(N, HP):
  NP = _pad_rows(N)
  RPT = NP // NS
  mesh = plsc.VectorSubcoreMesh(core_axis_name="c", subcore_axis_name="s")

  @functools.partial(
      pl.kernel,
      mesh=mesh,
      compiler_params=pltpu.CompilerParams(needs_layout_passes=False),
      out_type=jax.ShapeDtypeStruct((NC, NP, HP), jnp.float32),
      scratch_types=(
          [pltpu.VMEM((_EC,), jnp.int32) for _ in range(2 * _K)]
          + [pltpu.VMEM((_EC, HP), jnp.float32) for _ in range(_K)]
          + [pltpu.VMEM((2 * NW * 16,), jnp.int32)]
          + [pltpu.SemaphoreType.DMA((_K,)) for _ in range(3)]
          + [pltpu.VMEM_SHARED((NP, HP), jnp.float32)]
      ),
  )
  def agg_kernel(mpk_hbm, gidx_hbm, gdst_hbm, cnt_hbm, zeros_hbm, out_hbm,
                 *scratch):
    srcs = scratch[:_K]
    dsts = scratch[_K:2 * _K]
    rows = scratch[2 * _K:3 * _K]
    cbuf = scratch[3 * _K]
    isem, gsem, ssem = scratch[3 * _K + 1:3 * _K + 4]
    acc = scratch[3 * _K + 4]
    cid = lax.axis_index("c")
    sid = lax.axis_index("s")
    pltpu.sync_copy(zeros_hbm, rows[0])

    def zchunk(j, carry):
      pltpu.sync_copy(rows[0], acc.at[pl.ds(sid * RPT + j * _EC, _EC)])
      return carry

    lax.fori_loop(0, RPT // _EC, zchunk, 0)
    pltpu.sync_copy(cnt_hbm, cbuf)
    plsc.subcore_barrier()

    for reg in range(2):
      roff = cid * NW + sid * 2 + reg
      nouter = cbuf[pl.ds(roff * 16, 16)][0] // _PK
      base0 = roff * _CAP

      def outer(t, carry, base0=base0):
        base = base0 + t * _PK
        for j in range(_K):
          @pl.when(t > 0)
          def _drain(j=j):
            pltpu.make_async_copy(rows[j], acc.at[dsts[j]], ssem.at[j]).wait()
          pltpu.async_copy(gidx_hbm.at[pl.ds(base + j * _EC, _EC)], srcs[j],
                           isem.at[j])
          pltpu.async_copy(gdst_hbm.at[pl.ds(base + j * _EC, _EC)], dsts[j],
                           gsem.at[j])
        for j in range(_K):
          pltpu.make_async_copy(gidx_hbm.at[pl.ds(base + j * _EC, _EC)],
                                srcs[j], isem.at[j]).wait()
          pltpu.async_copy(mpk_hbm.at[srcs[j]], rows[j], isem.at[j])
        for j in range(_K):
          pltpu.make_async_copy(gdst_hbm.at[pl.ds(base + j * _EC, _EC)],
                                dsts[j], gsem.at[j]).wait()
          pltpu.make_async_copy(mpk_hbm.at[srcs[j]], rows[j],
                                isem.at[j]).wait()
          pltpu.async_copy(rows[j], acc.at[dsts[j]], ssem.at[j], add=True)
        return carry

      lax.fori_loop(0, nouter, outer, 0)

      @pl.when(nouter > 0)
      def _epilogue():
        for j in range(_K):
          pltpu.make_async_copy(rows[j], acc.at[dsts[j]], ssem.at[j]).wait()

    plsc.subcore_barrier()

    def wchunk(j, carry):
      pltpu.sync_copy(acc.at[pl.ds(sid * RPT + j * _EC, _EC)], rows[0])
      pltpu.sync_copy(rows[0], out_hbm.at[cid, pl.ds(sid * RPT + j * _EC, _EC)])
      return carry

    lax.fori_loop(0, RPT // _EC, wchunk, 0)

  return agg_kernel


def _tc_prep(x, w_first, deg_p):
  N = x.shape[0]
  H = w_first.shape[1]

  def body(x_ref, w_ref, dp_ref, dinv_ref, mp_ref):
    dp = dp_ref[...]
    deg = dp[0, :N, 0:1] + dp[1, :N, 0:1] + 1.0
    dinv = lax.rsqrt(deg)
    m = jnp.dot(x_ref[...], w_ref[...], preferred_element_type=jnp.float32)
    dinv_ref[...] = dinv
    mp_ref[...] = dinv * m

  return pl.pallas_call(
      body,
      out_shape=[
          jax.ShapeDtypeStruct((N, 1), jnp.float32),
          jax.ShapeDtypeStruct((N, H), jnp.float32),
      ],
  )(x, w_first, deg_p)


def _tc_layer(part, mp, dinv, b, gamma, beta, w_next, identity):
  N, H = mp.shape
  has_res = identity is not None

  def body(*refs):
    if has_res:
      (part_ref, mp_ref, dinv_ref, b_ref, g_ref, be_ref, w_ref, id_ref,
       h_ref, mpn_ref) = refs
    else:
      (part_ref, mp_ref, dinv_ref, b_ref, g_ref, be_ref, w_ref,
       h_ref, mpn_ref) = refs
    p = part_ref[...]
    dinv = dinv_ref[...]
    agg = p[0, :N, :H] + p[1, :N, H:2 * H] + mp_ref[...]
    a = jnp.maximum(dinv * agg + b_ref[...], 0.0)
    mu = jnp.mean(a, axis=0, keepdims=True)
    var = jnp.mean(a * a, axis=0, keepdims=True) - mu * mu
    h = (a - mu) * lax.rsqrt(var + 1e-5) * g_ref[...] + be_ref[...]
    if has_res:
      h = h + id_ref[...]
    h_ref[...] = h
    mpn_ref[...] = dinv * jnp.dot(h, w_ref[...],
                                  preferred_element_type=jnp.float32)

  args = [part, mp, dinv, b.reshape(1, H), gamma.reshape(1, H),
          beta.reshape(1, H), w_next]
  if has_res:
    args.append(identity)
  return pl.pallas_call(
      body,
      out_shape=[
          jax.ShapeDtypeStruct((N, H), jnp.float32),
          jax.ShapeDtypeStruct((N, H), jnp.float32),
      ],
  )(*args)


def _tc_final(part, mp, dinv, b_last, batch2d, fc1_w, fc1_b, fc2_w, fc2_b, G):
  N, H = mp.shape
  HP = part.shape[2]
  OUT = fc2_w.shape[1]
  BR = 1000
  nb = N // BR

  def body(part_ref, mp_ref, dinv_ref, b_ref, bt_ref, w1_ref, b1_ref,
           w2_ref, b2_ref, out_ref, ssum, smax, scnt):
    i = pl.program_id(0)

    @pl.when(i == 0)
    def _init():
      ssum[...] = jnp.zeros((G, H), jnp.float32)
      smax[...] = jnp.zeros((G, H), jnp.float32)
      scnt[...] = jnp.zeros((G, 1), jnp.float32)

    @pl.when(i < nb)
    def _block():
      p = part_ref[...]
      agg = p[0, :, :H] + p[1, :, H:2 * H] + mp_ref[...]
      h = jnp.maximum(dinv_ref[...] * agg + b_ref[...], 0.0)
      bt = bt_ref[...]
      gids = lax.broadcasted_iota(jnp.int32, (1, G), 1)
      onehot = (bt == gids).astype(jnp.float32)
      ssum[...] += lax.dot_general(onehot, h, (((0,), (0,)), ((), ())),
                                   preferred_element_type=jnp.float32)
      scnt[...] += lax.dot_general(onehot, jnp.ones((BR, 1), jnp.float32),
                                   (((0,), (0,)), ((), ())),
                                   preferred_element_type=jnp.float32)
      rows = []
      for g in range(G):
        rows.append(jnp.max(jnp.where(bt == g, h, 0.0), axis=0, keepdims=True))
      smax[...] = jnp.maximum(smax[...], jnp.concatenate(rows, axis=0))

    @pl.when(i == nb)
    def _head():
      cnt = scnt[...]
      mean = ssum[...] / jnp.maximum(cnt, 1.0)
      xg = jnp.concatenate([mean, smax[...]], axis=1)
      o = jnp.maximum(
          jnp.dot(xg, w1_ref[...], preferred_element_type=jnp.float32)
          + b1_ref[...], 0.0)
      out_ref[...] = (
          jnp.dot(o, w2_ref[...], preferred_element_type=jnp.float32)
          + b2_ref[...])

  blk = lambda i: (0, jnp.minimum(i, nb - 1), 0)
  rowblk = lambda i: (jnp.minimum(i, nb - 1), 0)
  zero2 = lambda i: (0, 0)
  return pl.pallas_call(
      body,
      grid=(nb + 1,),
      in_specs=[
          pl.BlockSpec((2, BR, HP), blk),
          pl.BlockSpec((BR, H), rowblk),
          pl.BlockSpec((BR, 1), rowblk),
          pl.BlockSpec((1, H), zero2),
          pl.BlockSpec((BR, 1), rowblk),
          pl.BlockSpec((2 * H, H), zero2),
          pl.BlockSpec((1, H), zero2),
          pl.BlockSpec((H, OUT), zero2),
          pl.BlockSpec((1, OUT), zero2),
      ],
      out_specs=pl.BlockSpec((G, OUT), zero2),
      scratch_shapes=[
          pltpu.VMEM((G, H), jnp.float32),
          pltpu.VMEM((G, H), jnp.float32),
          pltpu.VMEM((G, 1), jnp.float32),
      ],
      out_shape=jax.ShapeDtypeStruct((G, OUT), jnp.float32),
  )(part[:, :N, :], mp, dinv, b_last.reshape(1, H), batch2d, fc1_w,
    fc1_b.reshape(1, H), fc2_w, fc2_b.reshape(1, OUT))


def kernel(x, edge_index, batch, w_first, b_first, w_mid, b_mid, w_last,
           b_last, bn_gamma, bn_beta, fc1_w, fc1_b, fc2_w, fc2_b):
  N = x.shape[0]
  E = edge_index.shape[1]
  H = w_first.shape[1]
  HP = 2 * H
  G = 64

  src = edge_index[0].astype(jnp.int32)
  dst = edge_index[1].astype(jnp.int32)
  batch2d = batch.astype(jnp.int32).reshape(N, 1)

  zeros2 = jnp.zeros((_EC, HP), jnp.float32)
  zpad = jnp.zeros((8, HP), jnp.float32)

  gidx, gdst, cnts = _make_sc_part(E, N)(src, dst)
  agg = _make_sc_agg(N, HP)

  ones_pk = jnp.concatenate([jnp.ones((N // 2, HP), jnp.float32), zpad])
  deg_p = agg(ones_pk, gidx, gdst, cnts, zeros2)
  dinv, mp = _tc_prep(x, w_first, deg_p)

  nmid = w_mid.shape[0]
  h = None
  for j in range(nmid + 1):
    mpk = jnp.concatenate([mp.reshape(N // 2, HP), zpad])
    part = agg(mpk, gidx, gdst, cnts, zeros2)
    if j == 0:
      b, g, be, res = b_first, bn_gamma[0], bn_beta[0], None
    else:
      b, g, be, res = b_mid[j - 1], bn_gamma[j], bn_beta[j], h
    w_next = w_mid[j] if j < nmid else w_last
    h, mp = _tc_layer(part, mp, dinv, b, g, be, w_next, res)

  mpk = jnp.concatenate([mp.reshape(N // 2, HP), zpad])
  part = agg(mpk, gidx, gdst, cnts, zeros2)
  return _tc_final(part, mp, dinv, b_last, batch2d, fc1_w, fc1_b,
                   fc2_w, fc2_b, G)


# speedup vs baseline: 1.2886x; 1.2886x over previous
import functools

import jax
import jax.numpy as jnp
from jax import lax
from jax.experimental import pallas as pl
from jax.experimental.pallas import tpu as pltpu
from jax.experimental.pallas import tpu_sc as plsc

NC = 2
NS = 16
NW = NC * NS

_EC = 40
_K = 5


def _pad_rows(n):
  per = -(-n // NS)
  per = -(-per // 16) * 16
  return per * NS


@functools.lru_cache(maxsize=None)
def _make_sc_deg(E, N):
  NP = _pad_rows(N)
  RPT = NP // NS
  EW = E // NW
  NCH = EW // _EC
  mesh = plsc.VectorSubcoreMesh(core_axis_name="c", subcore_axis_name="s")

  @functools.partial(
      pl.kernel,
      mesh=mesh,
      out_type=jax.ShapeDtypeStruct((NC * NP,), jnp.float32),
      scratch_types=(
          [pltpu.VMEM((_EC,), jnp.int32) for _ in range(_K)]
          + [
              pltpu.VMEM((_EC,), jnp.float32),
              pltpu.VMEM((RPT,), jnp.float32),
              pltpu.SemaphoreType.DMA((_K,)),
              pltpu.SemaphoreType.DMA((_K,)),
              pltpu.VMEM_SHARED((NP,), jnp.float32),
          ]
      ),
  )
  def deg_kernel(dst_hbm, ones_hbm, out_hbm, *scratch):
    dsts = scratch[:_K]
    ones_v, zbuf, isem, ssem, acc = scratch[_K:]
    cid = lax.axis_index("c")
    sid = lax.axis_index("s")
    wid = sid * NC + cid
    zero16 = jnp.zeros((16,), jnp.float32)

    def zstore(i, carry):
      zbuf[pl.ds(i * 16, 16)] = zero16
      return carry

    lax.fori_loop(0, RPT // 16, zstore, 0)
    pltpu.sync_copy(zbuf, acc.at[pl.ds(sid * RPT, RPT)])
    pltpu.sync_copy(ones_hbm, ones_v)
    plsc.subcore_barrier()

    def outer(t, carry):
      base = wid * EW + t * (_K * _EC)
      for j in range(_K):
        @pl.when(t > 0)
        def _drain(j=j):
          pltpu.make_async_copy(ones_v, acc.at[dsts[j]], ssem.at[j]).wait()
        pltpu.async_copy(dst_hbm.at[pl.ds(base + j * _EC, _EC)], dsts[j],
                         isem.at[j])
      for j in range(_K):
        pltpu.make_async_copy(dst_hbm.at[pl.ds(base + j * _EC, _EC)], dsts[j],
                              isem.at[j]).wait()
        pltpu.async_copy(ones_v, acc.at[dsts[j]], ssem.at[j], add=True)
      return carry

    lax.fori_loop(0, NCH // _K, outer, 0)
    for j in range(_K):
      pltpu.make_async_copy(ones_v, acc.at[dsts[j]], ssem.at[j]).wait()
    plsc.subcore_barrier()
    pltpu.sync_copy(acc.at[pl.ds(sid * RPT, RPT)], zbuf)
    pltpu.sync_copy(zbuf, out_hbm.at[pl.ds(cid * NP + sid * RPT, RPT)])

  return deg_kernel


@functools.lru_cache(maxsize=None)
def _make_sc_agg(E, N, HP):
  NP = _pad_rows(N)
  RPT = NP // NS
  EW = E // NW
  NCH = EW // _EC
  mesh = plsc.VectorSubcoreMesh(core_axis_name="c", subcore_axis_name="s")

  @functools.partial(
      pl.kernel,
      mesh=mesh,
      out_type=jax.ShapeDtypeStruct((NC, NP, HP), jnp.float32),
      scratch_types=(
          [pltpu.VMEM((_EC,), jnp.int32) for _ in range(2 * _K)]
          + [pltpu.VMEM((_EC, HP), jnp.float32) for _ in range(_K)]
          + [pltpu.SemaphoreType.DMA((_K,)) for _ in range(3)]
          + [pltpu.VMEM_SHARED((NP, HP), jnp.float32)]
      ),
  )
  def agg_kernel(mp_hbm, src_hbm, dst_hbm, zeros_hbm, out_hbm, *scratch):
    srcs = scratch[:_K]
    dsts = scratch[_K:2 * _K]
    rows = scratch[2 * _K:3 * _K]
    isem, gsem, ssem = scratch[3 * _K:3 * _K + 3]
    acc = scratch[3 * _K + 3]
    cid = lax.axis_index("c")
    sid = lax.axis_index("s")
    wid = sid * NC + cid
    pltpu.sync_copy(zeros_hbm, rows[0])

    def zchunk(j, carry):
      pltpu.sync_copy(rows[0], acc.at[pl.ds(sid * RPT + j * _EC, _EC)])
      return carry

    lax.fori_loop(0, RPT // _EC, zchunk, 0)
    plsc.subcore_barrier()

    def outer(t, carry):
      base = wid * EW + t * (_K * _EC)
      for j in range(_K):
        @pl.when(t > 0)
        def _drain(j=j):
          pltpu.make_async_copy(rows[j], acc.at[dsts[j]], ssem.at[j]).wait()
        pltpu.async_copy(src_hbm.at[pl.ds(base + j * _EC, _EC)], srcs[j],
                         isem.at[j])
        pltpu.async_copy(dst_hbm.at[pl.ds(base + j * _EC, _EC)], dsts[j],
                         gsem.at[j])
      for j in range(_K):
        pltpu.make_async_copy(src_hbm.at[pl.ds(base + j * _EC, _EC)], srcs[j],
                              isem.at[j]).wait()
        pltpu.async_copy(mp_hbm.at[srcs[j]], rows[j], isem.at[j])
      for j in range(_K):
        pltpu.make_async_copy(dst_hbm.at[pl.ds(base + j * _EC, _EC)], dsts[j],
                              gsem.at[j]).wait()
        pltpu.make_async_copy(mp_hbm.at[srcs[j]], rows[j], isem.at[j]).wait()
        pltpu.async_copy(rows[j], acc.at[dsts[j]], ssem.at[j], add=True)
      return carry

    lax.fori_loop(0, NCH // _K, outer, 0)
    for j in range(_K):
      pltpu.make_async_copy(rows[j], acc.at[dsts[j]], ssem.at[j]).wait()
    plsc.subcore_barrier()

    NWCH = RPT // _EC
    for j in range(NWCH):
      b = j % 2
      if j >= 2:
        pltpu.make_async_copy(
            rows[b], out_hbm.at[cid, pl.ds(sid * RPT + (j - 2) * _EC, _EC)],
            gsem.at[b]).wait()
      pltpu.sync_copy(acc.at[pl.ds(sid * RPT + j * _EC, _EC)], rows[b])
      pltpu.async_copy(rows[b],
                       out_hbm.at[cid, pl.ds(sid * RPT + j * _EC, _EC)],
                       gsem.at[b])
    for j in range(NWCH - 2, NWCH):
      b = j % 2
      pltpu.make_async_copy(
          rows[b], out_hbm.at[cid, pl.ds(sid * RPT + j * _EC, _EC)],
          gsem.at[b]).wait()

  return agg_kernel


def _tc_prep(x, w_first, deg_p, HP):
  N = x.shape[0]
  H = w_first.shape[1]

  def body(x_ref, w_ref, dp_ref, dinv_ref, mp_ref):
    dp = dp_ref[...]
    deg = dp[0, :N, :] + dp[1, :N, :] + 1.0
    dinv = lax.rsqrt(deg)
    m = jnp.dot(x_ref[...], w_ref[...], preferred_element_type=jnp.float32)
    dinv_ref[...] = dinv
    mp_ref[...] = jnp.concatenate(
        [dinv * m, jnp.zeros((N, HP - H), jnp.float32)], axis=1)

  return pl.pallas_call(
      body,
      out_shape=[
          jax.ShapeDtypeStruct((N, 1), jnp.float32),
          jax.ShapeDtypeStruct((N, HP), jnp.float32),
      ],
  )(x, w_first, deg_p)


def _tc_layer(part, mp, dinv, b, gamma, beta, w_next, identity):
  N, HP = mp.shape
  H = w_next.shape[0]
  has_res = identity is not None

  def body(*refs):
    if has_res:
      (part_ref, mp_ref, dinv_ref, b_ref, g_ref, be_ref, w_ref, id_ref,
       h_ref, mpn_ref) = refs
    else:
      (part_ref, mp_ref, dinv_ref, b_ref, g_ref, be_ref, w_ref,
       h_ref, mpn_ref) = refs
    p = part_ref[...]
    dinv = dinv_ref[...]
    agg = p[0, :N, :H] + p[1, :N, :H] + mp_ref[..., :H]
    a = jnp.maximum(dinv * agg + b_ref[...], 0.0)
    mu = jnp.mean(a, axis=0, keepdims=True)
    var = jnp.mean(a * a, axis=0, keepdims=True) - mu * mu
    h = (a - mu) * lax.rsqrt(var + 1e-5) * g_ref[...] + be_ref[...]
    if has_res:
      h = h + id_ref[...]
    h_ref[...] = h
    mpn = dinv * jnp.dot(h, w_ref[...], preferred_element_type=jnp.float32)
    mpn_ref[...] = jnp.concatenate(
        [mpn, jnp.zeros((N, HP - H), jnp.float32)], axis=1)

  args = [part, mp, dinv, b.reshape(1, H), gamma.reshape(1, H),
          beta.reshape(1, H), w_next]
  if has_res:
    args.append(identity)
  return pl.pallas_call(
      body,
      out_shape=[
          jax.ShapeDtypeStruct((N, H), jnp.float32),
          jax.ShapeDtypeStruct((N, HP), jnp.float32),
      ],
  )(*args)


def _tc_final(part, mp, dinv, b_last, batch2d, fc1_w, fc1_b, fc2_w, fc2_b, G):
  N, HP = mp.shape
  H = b_last.shape[0]
  OUT = fc2_w.shape[1]
  BR = 1000
  nb = N // BR

  def body(part_ref, mp_ref, dinv_ref, b_ref, bt_ref, w1_ref, b1_ref,
           w2_ref, b2_ref, out_ref, ssum, smax, scnt):
    i = pl.program_id(0)

    @pl.when(i == 0)
    def _init():
      ssum[...] = jnp.zeros((G, H), jnp.float32)
      smax[...] = jnp.zeros((G, H), jnp.float32)
      scnt[...] = jnp.zeros((G, 1), jnp.float32)

    @pl.when(i < nb)
    def _block():
      p = part_ref[...]
      agg = p[0, :, :H] + p[1, :, :H] + mp_ref[..., :H]
      h = jnp.maximum(dinv_ref[...] * agg + b_ref[...], 0.0)
      bt = bt_ref[...]
      gids = lax.broadcasted_iota(jnp.int32, (1, G), 1)
      onehot = (bt == gids).astype(jnp.float32)
      ssum[...] += lax.dot_general(onehot, h, (((0,), (0,)), ((), ())),
                                   preferred_element_type=jnp.float32)
      scnt[...] += lax.dot_general(onehot, jnp.ones((BR, 1), jnp.float32),
                                   (((0,), (0,)), ((), ())),
                                   preferred_element_type=jnp.float32)
      rows = []
      for g in range(G):
        rows.append(jnp.max(jnp.where(bt == g, h, 0.0), axis=0, keepdims=True))
      smax[...] = jnp.maximum(smax[...], jnp.concatenate(rows, axis=0))

    @pl.when(i == nb)
    def _head():
      cnt = scnt[...]
      mean = ssum[...] / jnp.maximum(cnt, 1.0)
      xg = jnp.concatenate([mean, smax[...]], axis=1)
      o = jnp.maximum(
          jnp.dot(xg, w1_ref[...], preferred_element_type=jnp.float32)
          + b1_ref[...], 0.0)
      out_ref[...] = (
          jnp.dot(o, w2_ref[...], preferred_element_type=jnp.float32)
          + b2_ref[...])

  blk = lambda i: (0, jnp.minimum(i, nb - 1), 0)
  rowblk = lambda i: (jnp.minimum(i, nb - 1), 0)
  zero2 = lambda i: (0, 0)
  return pl.pallas_call(
      body,
      grid=(nb + 1,),
      in_specs=[
          pl.BlockSpec((2, BR, HP), blk),
          pl.BlockSpec((BR, HP), rowblk),
          pl.BlockSpec((BR, 1), rowblk),
          pl.BlockSpec((1, H), zero2),
          pl.BlockSpec((BR, 1), rowblk),
          pl.BlockSpec((2 * H, H), zero2),
          pl.BlockSpec((1, H), zero2),
          pl.BlockSpec((H, OUT), zero2),
          pl.BlockSpec((1, OUT), zero2),
      ],
      out_specs=pl.BlockSpec((G, OUT), zero2),
      scratch_shapes=[
          pltpu.VMEM((G, H), jnp.float32),
          pltpu.VMEM((G, H), jnp.float32),
          pltpu.VMEM((G, 1), jnp.float32),
      ],
      out_shape=jax.ShapeDtypeStruct((G, OUT), jnp.float32),
  )(part[:, :N, :], mp, dinv, b_last.reshape(1, H), batch2d, fc1_w,
    fc1_b.reshape(1, H), fc2_w, fc2_b.reshape(1, OUT))


def kernel(x, edge_index, batch, w_first, b_first, w_mid, b_mid, w_last,
           b_last, bn_gamma, bn_beta, fc1_w, fc1_b, fc2_w, fc2_b):
  N = x.shape[0]
  E = edge_index.shape[1]
  H = w_first.shape[1]
  HP = 128
  G = 64
  NP = _pad_rows(N)

  src = edge_index[0].astype(jnp.int32)
  dst = edge_index[1].astype(jnp.int32)
  batch2d = batch.astype(jnp.int32).reshape(N, 1)

  RPT = NP // NS
  ones_c = jnp.ones((_EC,), jnp.float32)
  zeros2 = jnp.zeros((_EC, HP), jnp.float32)

  deg_p = _make_sc_deg(E, N)(dst, ones_c)
  dinv, mp = _tc_prep(x, w_first, deg_p.reshape(NC, NP, 1), HP)

  agg = _make_sc_agg(E, N, HP)
  nmid = w_mid.shape[0]
  h = None
  for j in range(nmid + 1):
    part = agg(mp, src, dst, zeros2)
    if j == 0:
      b, g, be, res = b_first, bn_gamma[0], bn_beta[0], None
    else:
      b, g, be, res = b_mid[j - 1], bn_gamma[j], bn_beta[j], h
    w_next = w_mid[j] if j < nmid else w_last
    h, mp = _tc_layer(part, mp, dinv, b, g, be, w_next, res)

  part = agg(mp, src, dst, zeros2)
  return _tc_final(part, mp, dinv, b_last, batch2d, fc1_w, fc1_b,
                   fc2_w, fc2_b, G)
